# baseline (speedup 1.0000x reference)
import functools

import jax
import jax.numpy as jnp
import numpy as np
from jax import lax
from jax.experimental import pallas as pl
from jax.experimental.pallas import tpu as pltpu
from jax.experimental.pallas import tpu_sc as plsc

_M = 512
_PAIRS = _M * (_M - 1) // 2
_RPC = 511
_BLK = 8 * _RPC
_NG = 16
_NC = 4


def _merge_matrices() -> tuple[np.ndarray, np.ndarray]:
    la = np.zeros((512, 1022), dtype=np.float32)
    lb = np.zeros((512, 1022), dtype=np.float32)
    r = np.arange(511)
    la[r, 2 * r] = 1.0
    lb[r, 2 * r + 1] = 1.0
    return la, lb


def _pack_matrix() -> np.ndarray:
    p = np.zeros((128, 64), dtype=np.float32)
    u = np.arange(64)
    p[2 * u, u] = 1.0
    p[2 * u + 1, u] = 65536.0
    return p


def _start_copies(a_hbm, b_hbm, bufs, sems, step, slot):
    for ch in range(2):
        pltpu.make_async_copy(
            a_hbm.at[pl.ds(step * 8, 8), :, ch, 0, :],
            bufs[ch].at[slot], sems.at[slot, ch]).start()
        pltpu.make_async_copy(
            b_hbm.at[pl.ds(step * 8, 8), :, ch, 0, :],
            bufs[2 + ch].at[slot], sems.at[slot, 2 + ch]).start()


def _wait_copies(a_hbm, b_hbm, bufs, sems, step, slot):
    for ch in range(2):
        pltpu.make_async_copy(
            a_hbm.at[pl.ds(step * 8, 8), :, ch, 0, :],
            bufs[ch].at[slot], sems.at[slot, ch]).wait()
        pltpu.make_async_copy(
            b_hbm.at[pl.ds(step * 8, 8), :, ch, 0, :],
            bufs[2 + ch].at[slot], sems.at[slot, 2 + ch]).wait()


def _phase1_body(a_hbm, b_hbm, la_ref, lb_ref, p_ref, o_ref,
                 a0b, a1b, b0b, b1b, sems):
    b = pl.program_id(0)
    slot = lax.rem(b, 2)
    bufs = (a0b, a1b, b0b, b1b)

    @pl.when(b == 0)
    def _():
        _start_copies(a_hbm, b_hbm, bufs, sems, 0, 0)

    @pl.when(b < 7)
    def _():
        _start_copies(a_hbm, b_hbm, bufs, sems, b + 1, 1 - slot)

    _wait_copies(a_hbm, b_hbm, bufs, sems, b, slot)

    s0 = a0b[slot] + b0b[slot]
    s1 = a1b[slot] + b1b[slot]
    x = (s0 >= s1).astype(jnp.bfloat16)
    for gamma in range(2):
        g4 = 4 * gamma
        xs = x[g4] + x[g4 + 1] + x[g4 + 2] + x[g4 + 3]
        ya = jnp.dot(la_ref[...], xs, preferred_element_type=jnp.float32)
        yb = jnp.dot(lb_ref[...], xs, preferred_element_type=jnp.float32)
        pa = jnp.dot(ya.astype(jnp.bfloat16), p_ref[...],
                     preferred_element_type=jnp.float32)
        pb = jnp.dot(yb.astype(jnp.bfloat16), p_ref[...],
                     preferred_element_type=jnp.float32)
        packed = jnp.concatenate([pa, pb], axis=1).astype(jnp.int32)
        o_ref[pl.ds(gamma * 512, 512), :] = packed


def _phase1(a5, b5, la, lb, pmat):
    return pl.pallas_call(
        _phase1_body,
        grid=(8,),
        in_specs=[
            pl.BlockSpec(memory_space=pl.ANY),
            pl.BlockSpec(memory_space=pl.ANY),
            pl.BlockSpec((512, 1022), lambda b: (0, 0)),
            pl.BlockSpec((512, 1022), lambda b: (0, 0)),
            pl.BlockSpec((128, 64), lambda b: (0, 0)),
        ],
        out_specs=pl.BlockSpec((1024, 128), lambda b: (b, 0)),
        out_shape=jax.ShapeDtypeStruct((8192, 128), jnp.int32),
        scratch_shapes=[
            pltpu.VMEM((2, 8, 1022, 128), jnp.float32),
            pltpu.VMEM((2, 8, 1022, 128), jnp.float32),
            pltpu.VMEM((2, 8, 1022, 128), jnp.float32),
            pltpu.VMEM((2, 8, 1022, 128), jnp.float32),
            pltpu.SemaphoreType.DMA((2, 4)),
        ],
    )(a5, b5, la, lb, pmat)


_BATCH = 8


def _index_table() -> np.ndarray:
    i = np.arange(_M)[:, None]
    j = np.arange(_M)[None, :]
    lo = np.minimum(i, j)
    hi = np.maximum(i, j)
    p = lo * 511 - lo * (lo - 1) // 2 + hi - lo - 1
    p = np.where(i == j, 131071, p)
    return ((p >> 1) | ((p & 1) << 20)).astype(np.int32)


@functools.cache
def _sc_expand_fn():
    mesh = plsc.VectorSubcoreMesh(core_axis_name="c", subcore_axis_name="s")
    return pl.kernel(
        _sc_expand,
        mesh=mesh,
        out_type=jax.ShapeDtypeStruct((_NG, _M, _M), jnp.float32),
        scratch_types=[
            pltpu.VMEM((512, 128), jnp.int32),
            pltpu.VMEM((2, _BATCH, _M), jnp.float32),
            pltpu.VMEM((_BATCH, _M), jnp.float32),
            pltpu.VMEM((2, _BATCH, _M), jnp.int32),
            pltpu.SemaphoreType.DMA,
            pltpu.SemaphoreType.DMA,
        ],
        compiler_params=pltpu.CompilerParams(needs_layout_passes=False),
    )


def _sc_expand(xsp_hbm, xidx_hbm, out_hbm, plane_v, rows_v, zrow_v, idx_v,
               dsem, isem):
    g = lax.axis_index("s")
    h = lax.axis_index("c")
    pltpu.sync_copy(xsp_hbm.at[g], plane_v)
    nn = 512 - 32 * lax.rem(g, 8)
    base_row = h * 256
    jot = lax.iota(jnp.int32, 16)
    zero16 = jnp.zeros((16,), jnp.float32)
    nb = 256 // _BATCH

    def zinit(k, _):
        for r in range(_BATCH):
            zrow_v[r, pl.ds(k * 16, 16)] = zero16
        return 0

    lax.fori_loop(0, 32, zinit, 0)
    vb = jnp.clip((nn - base_row + _BATCH - 1) // _BATCH, 0, nb)

    def _idx_copy(bidx, slot):
        return pltpu.make_async_copy(
            xidx_hbm.at[pl.ds(base_row + bidx * _BATCH, _BATCH)],
            idx_v.at[slot], isem)

    def _drain():
        return pltpu.make_async_copy(
            rows_v.at[0], out_hbm.at[g, pl.ds(base_row, _BATCH)], dsem)

    _idx_copy(0, 0).start()

    def batch_body(bidx, _):
        i0 = base_row + bidx * _BATCH
        slot = lax.rem(bidx, 2)
        valid = i0 < nn

        @pl.when(valid)
        def _():
            @pl.when(bidx + 1 < vb)
            def _():
                _idx_copy(bidx + 1, 1 - slot).start()

            @pl.when(bidx >= 2)
            def _():
                _drain().wait()

            _idx_copy(bidx, slot).wait()

            def kloop(k, _):
                mcol = jnp.where(jot + k * 16 < nn, 1.0, 0.0)
                for r in range(_BATCH):
                    wsh = idx_v[slot, r, pl.ds(k * 16, 16)]
                    w_idx = wsh & 0xFFFF
                    w = plsc.load_gather(
                        plane_v,
                        [lax.shift_right_logical(w_idx, 7), w_idx & 127])
                    v = lax.shift_right_logical(
                        w, lax.shift_right_logical(wsh, 16)) & 0xFFFF
                    rows_v[slot, r, pl.ds(k * 16, 16)] = (
                        v.astype(jnp.float32) * mcol)
                return 0

            lax.fori_loop(0, 32, kloop, 0)
            pltpu.make_async_copy(
                rows_v.at[slot], out_hbm.at[g, pl.ds(i0, _BATCH)],
                dsem).start()

        @pl.when(jnp.logical_not(valid))
        def _():
            pltpu.sync_copy(zrow_v, out_hbm.at[g, pl.ds(i0, _BATCH)])

        return 0

    lax.fori_loop(0, nb, batch_body, 0)

    @pl.when(vb >= 1)
    def _():
        _drain().wait()

    @pl.when(vb >= 2)
    def _():
        _drain().wait()


def _as_tiles(x):
    return (x.reshape(64, 1022, 128, 2)
             .transpose(0, 1, 3, 2)
             .reshape(64, 1022, 2, 1, 128))


def kernel(adj_logits, gumbel_noise):
    a5 = _as_tiles(adj_logits)
    b5 = _as_tiles(gumbel_noise)
    la, lb = _merge_matrices()
    la = jnp.asarray(la, dtype=jnp.bfloat16)
    lb = jnp.asarray(lb, dtype=jnp.bfloat16)
    pmat = jnp.asarray(_pack_matrix(), dtype=jnp.bfloat16)
    xidx = jnp.asarray(_index_table())
    xsp = _phase1(a5, b5, la, lb, pmat)
    return _sc_expand_fn()(xsp.reshape(_NG, _M, 128), xidx)
